# baseline (speedup 1.0000x reference)
import functools

import jax
import jax.numpy as jnp
from jax import lax
from jax.experimental import pallas as pl
from jax.experimental.pallas import tpu as pltpu
from jax.experimental.pallas import tpu_sc as plsc

VOCAB = 100000
EMBED_DIM = 768
BATCH = 4
SEQ = 2048

NUM_CORES = 2
NUM_SUBCORES = 16
NW = NUM_CORES * NUM_SUBCORES
TOTAL = BATCH * SEQ
B_PER_W = TOTAL // NW
CHUNK = 32
NCHUNK = B_PER_W // CHUNK
NBUF = 5
WDELAY = 2


def _make_gather():
    mesh = plsc.VectorSubcoreMesh(core_axis_name="c", subcore_axis_name="s")

    @functools.partial(
        pl.kernel,
        mesh=mesh,
        out_type=jax.ShapeDtypeStruct((BATCH, SEQ, EMBED_DIM), jnp.float32),
        scratch_types=[
            pltpu.VMEM((B_PER_W,), jnp.int32),
        ] + [
            pltpu.VMEM((CHUNK, EMBED_DIM), jnp.float32) for _ in range(NBUF)
        ] + [
            pltpu.SemaphoreType.DMA,
            pltpu.SemaphoreType.DMA,
        ],
    )
    def k(ids_hbm, table_hbm, out_hbm, idx_v, *rest):
        bufs = rest[:NBUF]
        gsem, wsem = rest[NBUF], rest[NBUF + 1]
        wid = lax.axis_index("s") * NUM_CORES + lax.axis_index("c")
        wpb = SEQ // B_PER_W
        row = wid // wpb
        off = (wid % wpb) * B_PER_W
        pltpu.sync_copy(ids_hbm.at[row, pl.ds(off, B_PER_W)], idx_v)
        g = [None] * NBUF
        w = [None] * NBUF

        def gather(j, buf):
            return pltpu.async_copy(
                table_hbm.at[idx_v.at[pl.ds(j * CHUNK, CHUNK)]], buf, gsem)

        for j in range(min(NBUF, NCHUNK)):
            g[j] = gather(j, bufs[j])
        for j in range(NCHUNK):
            b = j % NBUF
            g[b].wait()
            w[b] = pltpu.async_copy(
                bufs[b], out_hbm.at[row, pl.ds(off + j * CHUNK, CHUNK)], wsem)
            jd = j - WDELAY
            if jd >= 0 and jd + NBUF < NCHUNK:
                bd = jd % NBUF
                w[bd].wait()
                w[bd] = None
                g[bd] = gather(jd + NBUF, bufs[bd])
        for b in range(NBUF):
            if w[b] is not None:
                w[b].wait()

    return k


_gather = _make_gather()


def kernel(encoder_input_ids, encoder_attention_mask, embed_table):
    ids = encoder_input_ids.astype(jnp.int32)
    hidden = _gather(ids, embed_table)
    return (encoder_input_ids, encoder_attention_mask, hidden)


# speedup vs baseline: 1.0070x; 1.0070x over previous
import functools

import jax
import jax.numpy as jnp
from jax import lax
from jax.experimental import pallas as pl
from jax.experimental.pallas import tpu as pltpu
from jax.experimental.pallas import tpu_sc as plsc

VOCAB = 100000
EMBED_DIM = 768
BATCH = 4
SEQ = 2048

NUM_CORES = 2
NUM_SUBCORES = 16
NW = NUM_CORES * NUM_SUBCORES
TOTAL = BATCH * SEQ
B_PER_W = TOTAL // NW
CHUNK = 32
NCHUNK = B_PER_W // CHUNK
NBUF = 5
WDELAY = 3


def _make_gather():
    mesh = plsc.VectorSubcoreMesh(core_axis_name="c", subcore_axis_name="s")

    @functools.partial(
        pl.kernel,
        mesh=mesh,
        out_type=jax.ShapeDtypeStruct((BATCH, SEQ, EMBED_DIM), jnp.float32),
        scratch_types=[
            pltpu.VMEM((B_PER_W,), jnp.int32),
        ] + [
            pltpu.VMEM((CHUNK, EMBED_DIM), jnp.float32) for _ in range(NBUF)
        ] + [
            pltpu.SemaphoreType.DMA,
            pltpu.SemaphoreType.DMA,
        ],
    )
    def k(ids_hbm, table_hbm, out_hbm, idx_v, *rest):
        bufs = rest[:NBUF]
        gsem, wsem = rest[NBUF], rest[NBUF + 1]
        wid = lax.axis_index("s") * NUM_CORES + lax.axis_index("c")
        wpb = SEQ // B_PER_W
        row = wid // wpb
        off = (wid % wpb) * B_PER_W
        pltpu.sync_copy(ids_hbm.at[row, pl.ds(off, B_PER_W)], idx_v)
        g = [None] * NBUF
        w = [None] * NBUF

        def gather(j, buf):
            return pltpu.async_copy(
                table_hbm.at[idx_v.at[pl.ds(j * CHUNK, CHUNK)]], buf, gsem)

        for j in range(min(NBUF, NCHUNK)):
            g[j] = gather(j, bufs[j])
        for j in range(NCHUNK):
            b = j % NBUF
            g[b].wait()
            w[b] = pltpu.async_copy(
                bufs[b], out_hbm.at[row, pl.ds(off + j * CHUNK, CHUNK)], wsem)
            jd = j - WDELAY
            if jd >= 0 and jd + NBUF < NCHUNK:
                bd = jd % NBUF
                w[bd].wait()
                w[bd] = None
                g[bd] = gather(jd + NBUF, bufs[bd])
        for b in range(NBUF):
            if w[b] is not None:
                w[b].wait()

    return k


_gather = _make_gather()


def kernel(encoder_input_ids, encoder_attention_mask, embed_table):
    ids = encoder_input_ids.astype(jnp.int32)
    hidden = _gather(ids, embed_table)
    return (encoder_input_ids, encoder_attention_mask, hidden)
